# baseline (speedup 1.0000x reference)
import functools

import jax
import jax.numpy as jnp
from jax import lax
from jax.experimental import pallas as pl
from jax.experimental.pallas import tpu as pltpu
from jax.experimental.pallas import tpu_sc as plsc

NC, NS = 2, 16
NW = NC * NS
CHUNK = 128
NBUF = 4

KROWS = 250
BLKS_PER_W = 125


def _mesh():
    return plsc.VectorSubcoreMesh(
        core_axis_name="c", subcore_axis_name="s",
        num_cores=NC, num_subcores=NS)


def _iota16():
    return lax.iota(jnp.int32, 16)


@functools.lru_cache(maxsize=None)
def _build_transpose(V: int, D: int):
    n_blk = V // KROWS

    @functools.partial(
        pl.kernel,
        out_type=jax.ShapeDtypeStruct((n_blk, KROWS, D), jnp.float32),
        mesh=_mesh(),
        scratch_types=[
            [pltpu.VMEM((D, KROWS), jnp.float32) for _ in range(2)],
            [pltpu.VMEM((KROWS, D), jnp.float32) for _ in range(2)],
            [pltpu.SemaphoreType.DMA for _ in range(2)],
            [pltpu.SemaphoreType.DMA for _ in range(2)],
        ],
        compiler_params=pltpu.CompilerParams(use_tc_tiling_on_sc=False, needs_layout_passes=False),
    )
    def k(tt_hbm, out_hbm, vbuf, obuf, ssem, wsem):
        wid = lax.axis_index("s") * NC + lax.axis_index("c")
        blk0 = wid * BLKS_PER_W
        iota = _iota16()

        def stage(j, jb):
            pltpu.async_copy(tt_hbm.at[:, blk0 + j], vbuf[jb], ssem[jb])

        def wait_stage(jb):
            pltpu.make_async_copy(
                tt_hbm.at[:, 0], vbuf[jb], ssem[jb]).wait()

        def writeback(j, jb):
            pltpu.async_copy(obuf[jb], out_hbm.at[blk0 + j], wsem[jb])

        def wait_writeback(jb):
            pltpu.make_async_copy(obuf[jb], out_hbm.at[blk0], wsem[jb]).wait()

        def transpose_block(jb):
            vb = vbuf[jb]
            ob = obuf[jb]

            def body_r(r, carry):
                rvec = jnp.full((16,), r, jnp.int32)
                for c0 in range(0, D, 16):
                    v = plsc.load_gather(vb, [c0 + iota, rvec])
                    ob[r, pl.ds(c0, 16)] = v
                return carry

            lax.fori_loop(0, KROWS, body_r, 0)

        stage(0, 0)
        stage(1, 1)

        for j in range(2):
            wait_stage(j)
            transpose_block(j)
            writeback(j, j)
            stage(j + 2, j)

        def body(o, carry):
            for p in range(2):
                j = 2 * o + p
                jb = p
                wait_stage(jb)
                wait_writeback(jb)
                transpose_block(jb)
                writeback(j, jb)
                stage(j + 2, jb)
            return carry

        lax.fori_loop(1, 61, body, 0)

        for j in range(122, BLKS_PER_W):
            jb = j % 2
            wait_stage(jb)
            wait_writeback(jb)
            transpose_block(jb)
            writeback(j, jb)
            if j + 2 < BLKS_PER_W:
                stage(j + 2, jb)
        wait_writeback((BLKS_PER_W - 1) % 2)
        wait_writeback(BLKS_PER_W % 2)

    return k


@functools.lru_cache(maxsize=None)
def _build_gather(n_chunk: int, D: int, B: int, H: int):
    @functools.partial(
        pl.kernel,
        out_type=jax.ShapeDtypeStruct((H, D, B), jnp.float32),
        mesh=_mesh(),
        scratch_types=[
            pltpu.VMEM((n_chunk, CHUNK), jnp.int32),
            [pltpu.VMEM((CHUNK, D), jnp.float32) for _ in range(NBUF)],
            [pltpu.VMEM((D, CHUNK), jnp.float32) for _ in range(2)],
            [pltpu.SemaphoreType.DMA for _ in range(NBUF)],
            [pltpu.SemaphoreType.DMA for _ in range(2)],
        ],
        compiler_params=pltpu.CompilerParams(use_tc_tiling_on_sc=False, needs_layout_passes=False),
    )
    def k(idx_hbm, table_hbm, out_hbm, idx_v, rows, tbuf, gsem, wsem):
        wid = lax.axis_index("s") * NC + lax.axis_index("c")
        base_row = wid * (n_chunk * CHUNK)
        iota = _iota16()

        pltpu.sync_copy(idx_hbm.at[wid], idx_v)

        def gather(g, b):
            pltpu.async_copy(table_hbm.at[idx_v.at[g]], rows[b], gsem[b])

        def wait_gather(b):
            pltpu.make_async_copy(
                table_hbm.at[idx_v.at[0]], rows[b], gsem[b]).wait()

        def writeback(g, t):
            flat = base_row + g * CHUNK
            h = flat // B
            b0 = flat % B
            pltpu.async_copy(
                tbuf[t], out_hbm.at[h, :, pl.ds(b0, CHUNK)], wsem[t])

        def wait_writeback(t):
            pltpu.make_async_copy(
                tbuf[t], out_hbm.at[0, :, pl.ds(0, CHUNK)], wsem[t]).wait()

        def transpose_chunk(b, t):
            rb = rows[b]
            tb = tbuf[t]

            def body_c(c, carry):
                cvec = jnp.full((16,), c, jnp.int32)
                for r0 in range(0, CHUNK, 16):
                    v = plsc.load_gather(rb, [r0 + iota, cvec])
                    tb[c, pl.ds(r0, 16)] = v
                return carry

            lax.fori_loop(0, D, body_c, 0)

        for g in range(NBUF):
            gather(g, g % NBUF)

        for g in range(2):
            wait_gather(g % NBUF)
            transpose_chunk(g % NBUF, g % 2)
            writeback(g, g % 2)
            gather(g + NBUF, g % NBUF)

        def body(o, carry):
            for p in range(4):
                g = 2 + 4 * o + p
                b = (2 + p) % NBUF
                t = p % 2
                wait_gather(b)
                wait_writeback(t)
                transpose_chunk(b, t)
                writeback(g, t)
                gather(g + NBUF, b)
            return carry

        lax.fori_loop(0, 48, body, 0)

        for g in range(194, n_chunk):
            b = g % NBUF
            t = g % 2
            wait_gather(b)
            wait_writeback(t)
            transpose_chunk(b, t)
            writeback(g, t)
            if g + NBUF < n_chunk:
                gather(g + NBUF, b)
        wait_writeback(0)
        wait_writeback(1)

    return k


def kernel(x, table):
    B, H = x.shape
    V, D = table.shape
    total = B * H
    n_chunk = total // (NW * CHUNK)

    tt = jnp.transpose(table).reshape(D, V // KROWS, KROWS)
    t_rm = _build_transpose(V, D)(tt)
    t_flat = t_rm.reshape(V, D)

    xt = jnp.transpose(x).astype(jnp.int32)
    idx = xt.reshape(NW, n_chunk, CHUNK)

    out_t = _build_gather(n_chunk, D, B, H)(idx, t_flat)
    return jnp.transpose(out_t, (2, 0, 1))


# speedup vs baseline: 2.8289x; 2.8289x over previous
import functools

import jax
import jax.numpy as jnp
from jax import lax
from jax.experimental import pallas as pl
from jax.experimental.pallas import tpu as pltpu
from jax.experimental.pallas import tpu_sc as plsc

NC, NS = 2, 16
NW = NC * NS
CHUNK = 128
NBUF = 8
AHEAD = 4


@functools.lru_cache(maxsize=None)
def _build(n_chunk: int, D: int):
    mesh = plsc.VectorSubcoreMesh(
        core_axis_name="c", subcore_axis_name="s",
        num_cores=NC, num_subcores=NS)

    @functools.partial(
        pl.kernel,
        out_type=jax.ShapeDtypeStruct((NW, n_chunk, D, 2 * D), jnp.float32),
        mesh=mesh,
        scratch_types=[
            pltpu.VMEM((n_chunk, CHUNK), jnp.int32),
            [pltpu.VMEM((CHUNK, D), jnp.float32) for _ in range(NBUF)],
            [pltpu.SemaphoreType.DMA for _ in range(NBUF)],
            [pltpu.SemaphoreType.DMA for _ in range(NBUF)],
        ],
        compiler_params=pltpu.CompilerParams(use_tc_tiling_on_sc=False),
    )
    def k(idx_hbm, table_hbm, out_hbm, idx_v, rows, gsem, wsem):
        wid = lax.axis_index("s") * NC + lax.axis_index("c")

        pltpu.sync_copy(idx_hbm.at[wid], idx_v)

        def gather(g, b):
            pltpu.async_copy(table_hbm.at[idx_v.at[g]], rows[b], gsem[b])

        def wait_gather(b):
            pltpu.make_async_copy(
                table_hbm.at[idx_v.at[0]], rows[b], gsem[b]).wait()

        def writeback(g, b):
            pltpu.async_copy(
                rows[b].at[pl.ds(0, D), :],
                out_hbm.at[wid, g, :, pl.ds(0, D)], wsem[b])
            pltpu.async_copy(
                rows[b].at[pl.ds(D, D), :],
                out_hbm.at[wid, g, :, pl.ds(D, D)], wsem[b])

        def wait_writeback(b):
            pltpu.make_async_copy(
                rows[b].at[pl.ds(0, D), :],
                out_hbm.at[wid, 0, :, pl.ds(0, D)], wsem[b]).wait()
            pltpu.make_async_copy(
                rows[b].at[pl.ds(D, D), :],
                out_hbm.at[wid, 0, :, pl.ds(D, D)], wsem[b]).wait()


        for g in range(AHEAD):
            gather(g, g % NBUF)

        for g in range(NBUF - AHEAD):
            wait_gather(g % NBUF)
            writeback(g, g % NBUF)
            gather(g + AHEAD, (g + AHEAD) % NBUF)

        steady0 = NBUF - AHEAD
        n_steady = (n_chunk - AHEAD) - steady0
        assert n_steady % NBUF == 0

        def body(outer, carry):
            for j in range(NBUF):
                g = steady0 + outer * NBUF + j
                b = (steady0 + j) % NBUF
                wait_gather(b)
                writeback(g, b)
                b2 = (steady0 + j + AHEAD) % NBUF
                wait_writeback(b2)
                gather(g + AHEAD, b2)
            return carry

        lax.fori_loop(0, n_steady // NBUF, body, 0)

        for g in range(n_chunk - AHEAD, n_chunk):
            wait_gather(g % NBUF)
            writeback(g, g % NBUF)
        for b in range(NBUF):
            wait_writeback(b)

    return k


def kernel(x, table):
    B, H = x.shape
    V, D = table.shape
    total = B * H
    n_chunk = total // (NW * CHUNK)
    idx = (x.reshape(NW, n_chunk, CHUNK // 2, 2)
           .transpose(0, 1, 3, 2)
           .reshape(NW, n_chunk, CHUNK)
           .astype(jnp.int32))
    out = _build(n_chunk, D)(idx, table)
    return out.reshape(B, H, D)


# speedup vs baseline: 4.1947x; 1.4828x over previous
import functools

import jax
import jax.numpy as jnp
from jax import lax
from jax.experimental import pallas as pl
from jax.experimental.pallas import tpu as pltpu
from jax.experimental.pallas import tpu_sc as plsc

NC, NS = 2, 16
NW = NC * NS
CHUNK = 128
NBUF = 4
AHEAD = 2


@functools.lru_cache(maxsize=None)
def _build(n_chunk: int, D: int):
    mesh = plsc.VectorSubcoreMesh(
        core_axis_name="c", subcore_axis_name="s",
        num_cores=NC, num_subcores=NS)

    @functools.partial(
        pl.kernel,
        out_type=jax.ShapeDtypeStruct((NW, n_chunk, CHUNK, 2 * D),
                                      jnp.float32),
        mesh=mesh,
        scratch_types=[
            pltpu.VMEM((n_chunk, CHUNK), jnp.int32),
            [pltpu.VMEM((CHUNK, D), jnp.float32) for _ in range(NBUF)],
            [pltpu.SemaphoreType.DMA for _ in range(NBUF)],
            [pltpu.SemaphoreType.DMA for _ in range(NBUF)],
        ],
        compiler_params=pltpu.CompilerParams(use_tc_tiling_on_sc=False),
    )
    def k(idx_hbm, table_hbm, out_hbm, idx_v, rows, gsem, wsem):
        wid = lax.axis_index("s") * NC + lax.axis_index("c")

        pltpu.sync_copy(idx_hbm.at[wid], idx_v)

        def gather(g, b):
            pltpu.async_copy(table_hbm.at[idx_v.at[g]], rows[b], gsem[b])

        def wait_gather(b):
            pltpu.make_async_copy(
                table_hbm.at[idx_v.at[0]], rows[b], gsem[b]).wait()

        def writeback(g, b):
            pltpu.async_copy(
                rows[b], out_hbm.at[wid, g, :, pl.ds(0, D)], wsem[b])

        def wait_writeback(b):
            pltpu.make_async_copy(
                rows[b], out_hbm.at[wid, 0, :, pl.ds(0, D)], wsem[b]).wait()

        for g in range(AHEAD):
            gather(g, g)

        def body(o, carry):
            for p in range(NBUF):
                g4 = o * NBUF + p

                @pl.when(g4 < n_chunk)
                def _():
                    wait_gather(p)
                    writeback(g4, p)

                b2 = (p + AHEAD) % NBUF
                g_next = g4 + AHEAD

                @pl.when(g_next < n_chunk)
                def _():
                    @pl.when(g_next >= NBUF)
                    def _():
                        wait_writeback(b2)
                    gather(g_next, b2)
            return carry

        lax.fori_loop(0, (n_chunk + NBUF - 1) // NBUF, body, 0)

        for b in range(NBUF):
            if n_chunk - NBUF + b >= 0:
                wait_writeback(b)

    return k


def kernel(x, table):
    B, H = x.shape
    V, D = table.shape
    total = B * H
    n_chunk = total // (NW * CHUNK)
    idx = x.reshape(NW, n_chunk, CHUNK).astype(jnp.int32)
    out = _build(n_chunk, D)(idx, table)
    return out[..., :D].reshape(B, H, D)


# speedup vs baseline: 4.1962x; 1.0004x over previous
import functools

import jax
import jax.numpy as jnp
from jax import lax
from jax.experimental import pallas as pl
from jax.experimental.pallas import tpu as pltpu
from jax.experimental.pallas import tpu_sc as plsc

NC, NS = 2, 16
NW = NC * NS
CHUNK = 128
NBUF = 4
AHEAD = 2


@functools.lru_cache(maxsize=None)
def _build(n_chunk: int, D: int):
    mesh = plsc.VectorSubcoreMesh(
        core_axis_name="c", subcore_axis_name="s",
        num_cores=NC, num_subcores=NS)

    @functools.partial(
        pl.kernel,
        out_type=jax.ShapeDtypeStruct((NW, n_chunk, CHUNK, 2 * D),
                                      jnp.float32),
        mesh=mesh,
        scratch_types=[
            pltpu.VMEM((n_chunk, CHUNK), jnp.int32),
            [pltpu.VMEM((CHUNK, D), jnp.float32) for _ in range(NBUF)],
            [pltpu.SemaphoreType.DMA for _ in range(NBUF)],
            [pltpu.SemaphoreType.DMA for _ in range(NBUF)],
        ],
        compiler_params=pltpu.CompilerParams(
            use_tc_tiling_on_sc=False,
            disable_bounds_checks=True,
            disable_semaphore_checks=True,
            skip_device_barrier=True,
        ),
    )
    def k(idx_hbm, table_hbm, out_hbm, idx_v, rows, gsem, wsem):
        wid = lax.axis_index("s") * NC + lax.axis_index("c")

        pltpu.sync_copy(idx_hbm.at[wid], idx_v)

        def gather(g, b):
            pltpu.async_copy(table_hbm.at[idx_v.at[g]], rows[b], gsem[b])

        def wait_gather(b):
            pltpu.make_async_copy(
                table_hbm.at[idx_v.at[0]], rows[b], gsem[b]).wait()

        def writeback(g, b):
            pltpu.async_copy(
                rows[b], out_hbm.at[wid, g, :, pl.ds(0, D)], wsem[b])

        def wait_writeback(b):
            pltpu.make_async_copy(
                rows[b], out_hbm.at[wid, 0, :, pl.ds(0, D)], wsem[b]).wait()

        for g in range(AHEAD):
            gather(g, g)

        def body(o, carry):
            for p in range(NBUF):
                g4 = o * NBUF + p

                @pl.when(g4 < n_chunk)
                def _():
                    wait_gather(p)
                    writeback(g4, p)

                b2 = (p + AHEAD) % NBUF
                g_next = g4 + AHEAD

                @pl.when(g_next < n_chunk)
                def _():
                    @pl.when(g_next >= NBUF)
                    def _():
                        wait_writeback(b2)
                    gather(g_next, b2)
            return carry

        lax.fori_loop(0, (n_chunk + NBUF - 1) // NBUF, body, 0)

        for b in range(NBUF):
            if n_chunk - NBUF + b >= 0:
                wait_writeback(b)

    return k


def kernel(x, table):
    B, H = x.shape
    V, D = table.shape
    total = B * H
    n_chunk = total // (NW * CHUNK)
    idx = x.reshape(NW, n_chunk, CHUNK).astype(jnp.int32)
    out = _build(n_chunk, D)(idx, table)
    return out[..., :D].reshape(B, H, D)


# speedup vs baseline: 4.1991x; 1.0007x over previous
import functools

import jax
import jax.numpy as jnp
from jax import lax
from jax.experimental import pallas as pl
from jax.experimental.pallas import tpu as pltpu
from jax.experimental.pallas import tpu_sc as plsc

NC, NS = 2, 16
NW = NC * NS
CHUNK = 128
NBUF = 4
AHEAD = 2


@functools.lru_cache(maxsize=None)
def _build(n_chunk: int, D: int):
    mesh = plsc.VectorSubcoreMesh(
        core_axis_name="c", subcore_axis_name="s",
        num_cores=NC, num_subcores=NS)

    @functools.partial(
        pl.kernel,
        out_type=jax.ShapeDtypeStruct((NW, n_chunk, CHUNK, 2 * D),
                                      jnp.float32),
        mesh=mesh,
        scratch_types=[
            pltpu.VMEM((n_chunk, CHUNK), jnp.int32),
            pltpu.VMEM((NBUF * CHUNK, D), jnp.float32),
            pltpu.SemaphoreType.DMA((NBUF,)),
            pltpu.SemaphoreType.DMA((NBUF,)),
        ],
        compiler_params=pltpu.CompilerParams(
            use_tc_tiling_on_sc=False,
            disable_bounds_checks=True,
            disable_semaphore_checks=True,
            skip_device_barrier=True,
        ),
    )
    def k(idx_hbm, table_hbm, out_hbm, idx_v, rows, gsem, wsem):
        wid = lax.axis_index("s") * NC + lax.axis_index("c")

        pltpu.sync_copy(idx_hbm.at[wid], idx_v)

        def rbuf(b):
            return rows.at[pl.ds(b * CHUNK, CHUNK), :]

        def gather(g, b):
            pltpu.async_copy(table_hbm.at[idx_v.at[g]], rbuf(b), gsem.at[b])

        def wait_gather(b):
            pltpu.make_async_copy(
                table_hbm.at[idx_v.at[0]], rbuf(b), gsem.at[b]).wait()

        def writeback(g, b):
            pltpu.async_copy(
                rbuf(b), out_hbm.at[wid, g, :, pl.ds(0, D)], wsem.at[b])

        def wait_writeback(b):
            pltpu.make_async_copy(
                rbuf(b), out_hbm.at[wid, 0, :, pl.ds(0, D)], wsem.at[b]).wait()

        def prime(g, carry):
            gather(g, g % NBUF)
            return carry

        lax.fori_loop(0, AHEAD, prime, 0)

        def body(g, carry):
            b = g % NBUF
            wait_gather(b)
            writeback(g, b)
            g_next = g + AHEAD
            b2 = g_next % NBUF

            @pl.when(g_next < n_chunk)
            def _():
                @pl.when(g_next >= NBUF)
                def _():
                    wait_writeback(b2)
                gather(g_next, b2)
            return carry

        lax.fori_loop(0, n_chunk, body, 0)

        def drain(b, carry):
            wait_writeback(b)
            return carry

        lax.fori_loop(0, NBUF, drain, 0)

    return k


def kernel(x, table):
    B, H = x.shape
    V, D = table.shape
    total = B * H
    n_chunk = total // (NW * CHUNK)
    idx = x.reshape(NW, n_chunk, CHUNK).astype(jnp.int32)
    out = _build(n_chunk, D)(idx, table)
    return out[..., :D].reshape(B, H, D)


# speedup vs baseline: 4.2034x; 1.0010x over previous
import functools

import jax
import jax.numpy as jnp
from jax import lax
from jax.experimental import pallas as pl
from jax.experimental.pallas import tpu as pltpu
from jax.experimental.pallas import tpu_sc as plsc

NC, NS = 2, 16
NW = NC * NS
CHUNK = 128
NBUF = 4
AHEAD = 2


@functools.lru_cache(maxsize=None)
def _build(n_chunk: int, D: int):
    mesh = plsc.VectorSubcoreMesh(
        core_axis_name="c", subcore_axis_name="s",
        num_cores=NC, num_subcores=NS)

    @functools.partial(
        pl.kernel,
        out_type=jax.ShapeDtypeStruct((NW, n_chunk, CHUNK, 2 * D),
                                      jnp.float32),
        mesh=mesh,
        scratch_types=[
            pltpu.VMEM((n_chunk, CHUNK), jnp.int32),
            pltpu.VMEM((NBUF * CHUNK, D), jnp.float32),
            pltpu.SemaphoreType.DMA((NBUF,)),
            pltpu.SemaphoreType.DMA((NBUF,)),
        ],
        compiler_params=pltpu.CompilerParams(use_tc_tiling_on_sc=False),
    )
    def k(idx_hbm, table_hbm, out_hbm, idx_v, rows, gsem, wsem):
        wid = lax.axis_index("s") * NC + lax.axis_index("c")

        pltpu.sync_copy(idx_hbm.at[wid], idx_v)

        def rbuf(b):
            return rows.at[pl.ds(b * CHUNK, CHUNK), :]

        def gather(g, b):
            pltpu.async_copy(table_hbm.at[idx_v.at[g]], rbuf(b), gsem.at[b])

        def wait_gather(b):
            pltpu.make_async_copy(
                table_hbm.at[idx_v.at[0]], rbuf(b), gsem.at[b]).wait()

        def writeback(g, b):
            pltpu.async_copy(
                rbuf(b), out_hbm.at[wid, g, :, pl.ds(0, D)], wsem.at[b])

        def wait_writeback(b):
            pltpu.make_async_copy(
                rbuf(b), out_hbm.at[wid, 0, :, pl.ds(0, D)], wsem.at[b]).wait()

        def prime(g, carry):
            gather(g, g % NBUF)
            return carry

        lax.fori_loop(0, AHEAD, prime, 0)

        def body(g, carry):
            b = g % NBUF
            wait_gather(b)
            writeback(g, b)
            g_next = g + AHEAD
            b2 = g_next % NBUF

            @pl.when(g_next < n_chunk)
            def _():
                @pl.when(g_next >= NBUF)
                def _():
                    wait_writeback(b2)
                gather(g_next, b2)
            return carry

        lax.fori_loop(0, n_chunk, body, 0)

        def drain(b, carry):
            wait_writeback(b)
            return carry

        lax.fori_loop(0, NBUF, drain, 0)

    return k


def kernel(x, table):
    B, H = x.shape
    V, D = table.shape
    total = B * H
    n_chunk = total // (NW * CHUNK)
    idx = x.reshape(NW, n_chunk, CHUNK).astype(jnp.int32)
    out = _build(n_chunk, D)(idx, table)
    return out[..., :D].reshape(B, H, D)
